# Initial kernel scaffold; baseline (speedup 1.0000x reference)
import jax
import jax.numpy as jnp
from jax.experimental import pallas as pl


def kernel(h, edge_index, W1, b1, W2, b2, W3, b3):
    raise NotImplementedError("write your pallas kernel here")


# speedup vs baseline: 3.8757x; 3.8757x over previous
import functools

import jax
import jax.numpy as jnp
from jax import lax
from jax.experimental import pallas as pl
from jax.experimental.pallas import tpu as pltpu, tpu_sc as plsc

N = 10000
E = 160000
D = 256
HALF = 128

NC = 2
NS = 16
CHUNK = 128
NCH = 79
EPT = CHUNK * NCH
E_PAD = NS * EPT
TRASH = N
N_ACC = 10112
ROWS_I = N_ACC // NS

BN = 2000
GRID_N = N // BN

_mesh = plsc.VectorSubcoreMesh(
    core_axis_name="c", subcore_axis_name="s", num_cores=NC, num_subcores=NS
)


@functools.partial(
    pl.kernel,
    out_type=jax.ShapeDtypeStruct((NC, N_ACC, HALF), jnp.float32),
    mesh=_mesh,
    scratch_types=[
        pltpu.VMEM((NCH, CHUNK), jnp.int32),
        pltpu.VMEM((CHUNK, HALF), jnp.float32),
        pltpu.VMEM_SHARED((N_ACC, HALF), jnp.float32),
    ],
)
def _sc_degrees(src_hbm, dst_hbm, ones_hbm, zeros_hbm, out_hbm, idx_v, ones_v, acc_sh):
    c = lax.axis_index("c")
    s = lax.axis_index("s")
    pltpu.sync_copy(zeros_hbm, acc_sh.at[pl.ds(s * ROWS_I, ROWS_I)])
    pltpu.sync_copy(ones_hbm, ones_v)

    @pl.when(c == 0)
    def _():
        pltpu.sync_copy(dst_hbm.at[s], idx_v)

    @pl.when(c == 1)
    def _():
        pltpu.sync_copy(src_hbm.at[s], idx_v)

    plsc.subcore_barrier()

    def body(j, _):
        pltpu.sync_copy(ones_v, acc_sh.at[idx_v.at[j]], add=True)
        return ()

    lax.fori_loop(0, NCH, body, ())
    plsc.subcore_barrier()
    pltpu.sync_copy(
        acc_sh.at[pl.ds(s * ROWS_I, ROWS_I)],
        out_hbm.at[c].at[pl.ds(s * ROWS_I, ROWS_I)],
    )


@functools.partial(
    pl.kernel,
    out_type=(
        jax.ShapeDtypeStruct((N_ACC, HALF), jnp.float32),
        jax.ShapeDtypeStruct((N_ACC, HALF), jnp.float32),
    ),
    mesh=_mesh,
    scratch_types=[
        pltpu.VMEM((NCH, CHUNK), jnp.int32),
        pltpu.VMEM((NCH, CHUNK), jnp.int32),
        pltpu.VMEM((CHUNK, HALF), jnp.float32),
        pltpu.VMEM_SHARED((N_ACC, HALF), jnp.float32),
        pltpu.SemaphoreType.DMA,
    ],
)
def _sc_spmv(x0, x1, src_hbm, dst_hbm, zeros_hbm, out0, out1,
             idx_s, idx_d, rows_v, acc_sh, sem):
    c = lax.axis_index("c")
    s = lax.axis_index("s")
    pltpu.sync_copy(zeros_hbm, acc_sh.at[pl.ds(s * ROWS_I, ROWS_I)])
    pltpu.sync_copy(src_hbm.at[s], idx_s)
    pltpu.sync_copy(dst_hbm.at[s], idx_d)
    plsc.subcore_barrier()

    def run(x_hbm):
        def body(j, _):
            pltpu.async_copy(x_hbm.at[idx_s.at[j]], rows_v, sem).wait()
            pltpu.sync_copy(rows_v, acc_sh.at[idx_d.at[j]], add=True)
            return ()

        lax.fori_loop(0, NCH, body, ())

    @pl.when(c == 0)
    def _():
        run(x0)

    @pl.when(c == 1)
    def _():
        run(x1)

    plsc.subcore_barrier()

    @pl.when(c == 0)
    def _():
        pltpu.sync_copy(acc_sh.at[pl.ds(s * ROWS_I, ROWS_I)],
                        out0.at[pl.ds(s * ROWS_I, ROWS_I)])

    @pl.when(c == 1)
    def _():
        pltpu.sync_copy(acc_sh.at[pl.ds(s * ROWS_I, ROWS_I)],
                        out1.at[pl.ds(s * ROWS_I, ROWS_I)])


def _norm(deg):
    return jnp.where(deg > 0, lax.rsqrt(jnp.maximum(deg, 1e-12)), 0.0)


def _tc_first_body(dego_ref, h_ref, w_ref, y0_ref, y1_ref):
    i = pl.program_id(0)
    norm = _norm(dego_ref[pl.ds(i * BN, BN), 0:1])
    y = jnp.dot(h_ref[...] * norm, w_ref[...],
                preferred_element_type=jnp.float32,
                precision=lax.Precision.HIGHEST)
    y0_ref[...] = y[:, :HALF]
    y1_ref[...] = y[:, HALF:]


def _tc_mid_body(degi_ref, dego_ref, a0_ref, a1_ref, b_ref, w_ref, y0_ref, y1_ref):
    i = pl.program_id(0)
    ni = _norm(degi_ref[pl.ds(i * BN, BN), 0:1])
    no = _norm(dego_ref[pl.ds(i * BN, BN), 0:1])
    a = jnp.concatenate([a0_ref[...], a1_ref[...]], axis=1)
    hh = jnp.maximum(a * ni + b_ref[...], 0.0)
    y = jnp.dot(hh * no, w_ref[...],
                preferred_element_type=jnp.float32,
                precision=lax.Precision.HIGHEST)
    y0_ref[...] = y[:, :HALF]
    y1_ref[...] = y[:, HALF:]


def _tc_final_body(degi_ref, a0_ref, a1_ref, b_ref, o_ref):
    i = pl.program_id(0)
    ni = _norm(degi_ref[pl.ds(i * BN, BN), 0:1])
    a = jnp.concatenate([a0_ref[...], a1_ref[...]], axis=1)
    hh = jnp.maximum(a * ni + b_ref[...], 0.0)
    part = jnp.sum(hh, axis=0, keepdims=True) * (1.0 / N)

    @pl.when(i == 0)
    def _():
        o_ref[...] = jnp.zeros_like(o_ref)

    o_ref[...] += part


_deg_spec = pl.BlockSpec((N, 16), lambda i: (0, 0))
_half_spec = pl.BlockSpec((BN, HALF), lambda i: (i, 0))
_half_pad_spec = pl.BlockSpec((BN, HALF), lambda i: (i, 0))
_full_spec = pl.BlockSpec((BN, D), lambda i: (i, 0))
_w_spec = pl.BlockSpec((D, D), lambda i: (0, 0))
_b_spec = pl.BlockSpec((1, D), lambda i: (0, 0))

_tc_first = pl.pallas_call(
    _tc_first_body,
    grid=(GRID_N,),
    in_specs=[_deg_spec, _full_spec, _w_spec],
    out_specs=[_half_spec, _half_spec],
    out_shape=[
        jax.ShapeDtypeStruct((N, HALF), jnp.float32),
        jax.ShapeDtypeStruct((N, HALF), jnp.float32),
    ],
)

_tc_mid = pl.pallas_call(
    _tc_mid_body,
    grid=(GRID_N,),
    in_specs=[_deg_spec, _deg_spec, _half_pad_spec, _half_pad_spec, _b_spec, _w_spec],
    out_specs=[_half_spec, _half_spec],
    out_shape=[
        jax.ShapeDtypeStruct((N, HALF), jnp.float32),
        jax.ShapeDtypeStruct((N, HALF), jnp.float32),
    ],
)

_tc_final = pl.pallas_call(
    _tc_final_body,
    grid=(GRID_N,),
    in_specs=[_deg_spec, _half_pad_spec, _half_pad_spec, _b_spec],
    out_specs=pl.BlockSpec((1, D), lambda i: (0, 0)),
    out_shape=jax.ShapeDtypeStruct((1, D), jnp.float32),
)


def kernel(h, edge_index, W1, b1, W2, b2, W3, b3):
    src = edge_index[0].astype(jnp.int32)
    dst = edge_index[1].astype(jnp.int32)
    pad = E_PAD - E
    src_g = jnp.concatenate([src, jnp.zeros((pad,), jnp.int32)]).reshape(NS, NCH, CHUNK)
    src_d = jnp.concatenate([src, jnp.full((pad,), TRASH, jnp.int32)]).reshape(NS, NCH, CHUNK)
    dst_p = jnp.concatenate([dst, jnp.full((pad,), TRASH, jnp.int32)]).reshape(NS, NCH, CHUNK)

    ones_rows = jnp.ones((CHUNK, HALF), jnp.float32)
    zeros_half = jnp.zeros((ROWS_I, HALF), jnp.float32)

    degs = _sc_degrees(src_d, dst_p, ones_rows, zeros_half)
    deg_in = degs[0][:N, :16]
    deg_out = degs[1][:N, :16]

    b1r = b1.reshape(1, D)
    b2r = b2.reshape(1, D)
    b3r = b3.reshape(1, D)

    y0, y1 = _tc_first(deg_out, h, W1)
    a0, a1 = _sc_spmv(y0, y1, src_g, dst_p, zeros_half)
    y0, y1 = _tc_mid(deg_in, deg_out, a0, a1, b1r, W2)
    a0, a1 = _sc_spmv(y0, y1, src_g, dst_p, zeros_half)
    y0, y1 = _tc_mid(deg_in, deg_out, a0, a1, b2r, W3)
    a0, a1 = _sc_spmv(y0, y1, src_g, dst_p, zeros_half)
    out = _tc_final(deg_in, a0, a1, b3r)
    return out.reshape(D)
